# baseline (speedup 1.0000x reference)
import functools

import numpy as np
import jax
import jax.numpy as jnp
from jax import lax
from jax.experimental import pallas as pl
from jax.experimental.pallas import tpu as pltpu
from jax.experimental.pallas import tpu_sc as plsc

_VOCAB = 100000
_VP = 100352
_B = 64
_D = 1024
_CH = 3584
_NCH = 28
_NC128 = 784
_NG16 = 6272
_C = 112
_EPS = 1e-5
_NEG = float("-inf")


def _threefry_uniform(n):
    rot = [[13, 15, 26, 6], [17, 29, 16, 24]]
    k1, k2 = np.uint32(0), np.uint32(1)
    ks = [k1, k2, np.uint32(k1 ^ k2 ^ np.uint32(0x1BD11BDA))]

    def rotl(v, r):
        return ((v << np.uint32(r)) | (v >> np.uint32(32 - r))).astype(np.uint32)

    x0 = np.zeros(n, np.uint32)
    x1 = np.arange(n, dtype=np.uint32)
    x0 = (x0 + ks[0]).astype(np.uint32)
    x1 = (x1 + ks[1]).astype(np.uint32)
    for i in range(5):
        for r in rot[i % 2]:
            x0 = (x0 + x1).astype(np.uint32)
            x1 = rotl(x1, r)
            x1 = (x1 ^ x0).astype(np.uint32)
        x0 = (x0 + ks[(i + 1) % 3]).astype(np.uint32)
        x1 = (x1 + ks[(i + 2) % 3] + np.uint32(i + 1)).astype(np.uint32)
    bits = (x0 ^ x1).astype(np.uint32)
    fb = (bits >> np.uint32(9)) | np.uint32(0x3F800000)
    fl = fb.view(np.float32) - np.float32(1.0)
    tiny = np.float32(np.finfo(np.float32).tiny)
    return np.maximum(tiny, (fl + tiny).astype(np.float32))


_U_CONST = np.pad(
    _threefry_uniform(_B * _VOCAB).reshape(_B, _VOCAB),
    ((0, 0), (0, _VP - _VOCAB)), constant_values=0.5)


def _sc_gather(table, idx, n_idx, width, rows_per_batch, n_workers,
               tc_tiling=True):
    mesh = plsc.VectorSubcoreMesh(core_axis_name="c", subcore_axis_name="s")
    per_w = n_idx // n_workers
    n_batch = per_w // rows_per_batch

    @functools.partial(
        pl.kernel, mesh=mesh,
        out_type=jax.ShapeDtypeStruct((n_idx, width), jnp.float32),
        scratch_types=[
            pltpu.VMEM((rows_per_batch,), jnp.int32),
            pltpu.VMEM((rows_per_batch, width), jnp.float32),
            pltpu.SemaphoreType.DMA,
        ],
        compiler_params=pltpu.CompilerParams(use_tc_tiling_on_sc=tc_tiling),
    )
    def k(table_hbm, idx_hbm, out_hbm, idx_v, rows_v, sem):
        wid = lax.axis_index("s") * 2 + lax.axis_index("c")

        @pl.when(wid < n_workers)
        def _():
            base = wid * per_w
            for b in range(n_batch):
                off = base + b * rows_per_batch
                pltpu.sync_copy(idx_hbm.at[pl.ds(off, rows_per_batch)], idx_v)
                pltpu.async_copy(table_hbm.at[idx_v], rows_v, sem).wait()
                pltpu.sync_copy(rows_v, out_hbm.at[pl.ds(off, rows_per_batch)])

    return k(table, idx)


def _k1_body(h_ref, emb_ref, t_ref, slog_ref, idx1_ref, m_ref, macc):
    lg = lax.dot_general(h_ref[...], emb_ref[...], (((1,), (1,)), ((), ())),
                         preferred_element_type=jnp.float32)
    slog = lg / t_ref[...]
    j = pl.program_id(0)
    gi = j * _CH + lax.broadcasted_iota(jnp.int32, (_B, _CH), 1)
    slog = jnp.where(gi < _VOCAB, slog, _NEG)
    slog_ref[...] = slog

    @pl.when(j == 0)
    def _():
        macc[...] = jnp.full((_B, 896), _NEG, jnp.float32)

    cm = jnp.max(slog.reshape(_B, 28, 128), axis=2)
    placed = pltpu.roll(
        jnp.concatenate(
            [cm, jnp.full((_B, 896 - 28), _NEG, jnp.float32)], axis=1),
        j * 28, axis=1)
    macc[...] = jnp.maximum(macc[...], placed)

    @pl.when(j == _NCH - 1)
    def _():
        vals = macc[...]
        m_ref[...] = jnp.max(vals, axis=1, keepdims=True)
        iota = lax.broadcasted_iota(jnp.int32, (_B, 896), 1)
        ci = lax.broadcasted_iota(jnp.int32, (_B, _C), 1)
        rbase = lax.broadcasted_iota(jnp.int32, (_B, 1), 0) * _NC128

        def body(i, carry):
            vals, idx1 = carry
            m = jnp.max(vals, axis=1, keepdims=True)
            idx = jnp.min(jnp.where(vals >= m, iota, jnp.int32(1 << 30)),
                          axis=1, keepdims=True)
            idx1 = jnp.where(ci == i, rbase + idx, idx1)
            vals = jnp.where(iota == idx, _NEG, vals)
            return vals, idx1

        _, idx1 = lax.fori_loop(0, _C, body,
                                (vals, jnp.zeros((_B, _C), jnp.int32)))
        idx1_ref[...] = idx1


def _k3_body(slog_ref, m_ref, g1_ref, cid_ref, s_ref, idx2_ref, acc):
    j = pl.program_id(0)
    e = jnp.exp(slog_ref[...] - m_ref[...])
    ps = jnp.sum(e, axis=1, keepdims=True)
    prev = jnp.where(j == 0, 0.0, acc[...])
    acc[...] = prev + ps
    s_ref[...] = acc[...]

    @pl.when(j == _NCH - 1)
    def _k3_last():
        _k3_extract(g1_ref, cid_ref, idx2_ref)


def _k3_extract(g1_ref, cid_ref, idx2_ref):
    x = g1_ref[...]
    m16 = jnp.max(x.reshape(_B, _C * 8, 16), axis=-1)
    cid = cid_ref[...] % _NC128
    iota = lax.broadcasted_iota(jnp.int32, (_B, _C * 8), 1)
    ci = lax.broadcasted_iota(jnp.int32, (_B, _C), 1)
    lane = lax.broadcasted_iota(jnp.int32, (_B, _C), 1)
    rbase = lax.broadcasted_iota(jnp.int32, (_B, 1), 0) * _NG16

    def body(i, carry):
        vals, idx2 = carry
        m = jnp.max(vals, axis=1, keepdims=True)
        g = jnp.min(jnp.where(vals >= m, iota, jnp.int32(1 << 30)),
                    axis=1, keepdims=True)
        c = g >> 3
        sub = g & 7
        cidg = jnp.max(jnp.where(lane == c, cid, jnp.int32(-1)),
                       axis=1, keepdims=True)
        idx2 = jnp.where(ci == i, rbase + cidg * 8 + sub, idx2)
        vals = jnp.where(iota == g, _NEG, vals)
        return vals, idx2

    _, idx2 = lax.fori_loop(0, _C, body, (m16, jnp.zeros((_B, _C), jnp.int32)))
    idx2_ref[...] = idx2


def _k4_decide(g2_ref, idx2_ref, m_ref, s_ref, tk_ref, tp_ref,
               lcut_ref, icut_ref, tot_ref):
    vals0 = g2_ref[...]
    base16 = (idx2_ref[...] % _NG16) * 16
    vidx = (jnp.broadcast_to(base16[:, :, None], (_B, _C, 16)).reshape(_B, _C * 16)
            + (lax.broadcasted_iota(jnp.int32, (_B, _C * 16), 1) & 15))
    ci = lax.broadcasted_iota(jnp.int32, (_B, _C), 1)

    def body(i, carry):
        vals, cv, cvi = carry
        m = jnp.max(vals, axis=1, keepdims=True)
        vi = jnp.min(jnp.where(vals >= m, vidx, jnp.int32(1 << 30)),
                     axis=1, keepdims=True)
        cv = jnp.where(ci == i, m, cv)
        cvi = jnp.where(ci == i, vi, cvi)
        vals = jnp.where((vals >= m) & (vidx == vi), _NEG, vals)
        return vals, cv, cvi

    _, cv, cvi = lax.fori_loop(
        0, _C, body,
        (vals0, jnp.full((_B, _C), _NEG, jnp.float32),
         jnp.zeros((_B, _C), jnp.int32)))

    p = jnp.exp(cv - m_ref[...]) / s_ref[...]
    tkm1 = tk_ref[...] - 1
    thresh = jnp.max(jnp.where(ci == tkm1, p, _NEG), axis=1, keepdims=True)
    sel = p >= thresh
    sp = jnp.where(sel, p, 0.0)
    cum = sp
    for d in (1, 2, 4, 8, 16, 32, 64):
        cum = cum + jnp.concatenate(
            [jnp.zeros((_B, d), jnp.float32), cum[:, :_C - d]], axis=1)
    shifted_cum = jnp.concatenate(
        [jnp.zeros((_B, 1), jnp.float32), cum[:, :_C - 1]], axis=1)
    kept = sel & jnp.logical_not(shifted_cum > tp_ref[...])
    tot_ref[...] = jnp.sum(jnp.where(kept, p, 0.0), axis=1, keepdims=True)
    lastpos = jnp.max(jnp.where(kept, ci, -1), axis=1, keepdims=True)
    lcut_ref[...] = jnp.max(jnp.where(ci == lastpos, cv, _NEG),
                            axis=1, keepdims=True)
    icut_ref[...] = jnp.max(jnp.where(ci == lastpos, cvi, -1),
                            axis=1, keepdims=True)


def _k5_body(slog_ref, u_ref, m_ref, s_ref, g2_ref, idx2_ref, tk_ref, tp_ref,
             out_ref, tok_ref, bestv, besti, lcut_ref, icut_ref, tot_ref):
    j = pl.program_id(0)

    @pl.when(j == 0)
    def _decide():
        _k4_decide(g2_ref, idx2_ref, m_ref, s_ref, tk_ref, tp_ref,
                   lcut_ref, icut_ref, tot_ref)
    x = slog_ref[...]
    gi = j * _CH + lax.broadcasted_iota(jnp.int32, (_B, _CH), 1)
    valid = gi < _VOCAB
    p = jnp.exp(x - m_ref[...]) / s_ref[...]
    keep = (x > lcut_ref[...]) | ((x == lcut_ref[...]) & (gi <= icut_ref[...]))
    outv = jnp.where(keep & valid, p / tot_ref[...], 0.0)
    out_ref[...] = outv

    g = -jnp.log(-jnp.log(u_ref[...]))
    val = jnp.where(valid, jnp.log(outv + 1e-20) + g, _NEG)
    mv = jnp.max(val, axis=1, keepdims=True)
    mi = jnp.min(jnp.where(val >= mv, gi, jnp.int32(1 << 30)),
                 axis=1, keepdims=True)
    pv = jnp.where(j == 0, _NEG, bestv[...])
    pi = jnp.where(j == 0, 0, besti[...])
    upd = mv > pv
    bestv[...] = jnp.where(upd, mv, pv)
    besti[...] = jnp.where(upd, mi, pi)
    tok_ref[...] = besti[...]


def kernel(hidden_states, embedding, last_token_indices, temperatures,
           top_ps, top_ks):
    lti = last_token_indices.astype(jnp.int32)
    temps_col = jnp.where(temperatures < _EPS, 1.0, temperatures)[:, None]
    tk_col = jnp.clip(top_ks, 1, _VOCAB).astype(jnp.int32)[:, None]
    tp_col = top_ps[:, None]
    u = jnp.asarray(_U_CONST)

    h = _sc_gather(hidden_states, lti, _B, _D, 8, 8)

    slog, idx1, m_col = pl.pallas_call(
        _k1_body,
        grid=(_NCH,),
        in_specs=[
            pl.BlockSpec((_B, _D), lambda j: (0, 0)),
            pl.BlockSpec((_CH, _D), lambda j: (j, 0)),
            pl.BlockSpec((_B, 1), lambda j: (0, 0)),
        ],
        out_specs=[
            pl.BlockSpec((_B, _CH), lambda j: (0, j)),
            pl.BlockSpec((_B, _C), lambda j: (0, 0)),
            pl.BlockSpec((_B, 1), lambda j: (0, 0)),
        ],
        out_shape=[
            jax.ShapeDtypeStruct((_B, _VP), jnp.float32),
            jax.ShapeDtypeStruct((_B, _C), jnp.int32),
            jax.ShapeDtypeStruct((_B, 1), jnp.float32),
        ],
        scratch_shapes=[pltpu.VMEM((_B, 896), jnp.float32)],
        compiler_params=pltpu.CompilerParams(
            dimension_semantics=("arbitrary",)),
    )(h, embedding, temps_col)

    slog_lin = slog.reshape(_B * _NC128, 128)
    g1 = _sc_gather(slog_lin, idx1.reshape(_B * _C),
                    _B * _C, 128, _C, 32)

    s_col, idx2 = pl.pallas_call(
        _k3_body,
        grid=(_NCH,),
        in_specs=[
            pl.BlockSpec((_B, _CH), lambda j: (0, j)),
            pl.BlockSpec((_B, 1), lambda j: (0, 0)),
            pl.BlockSpec((_B, _C * 128), lambda j: (0, 0)),
            pl.BlockSpec((_B, _C), lambda j: (0, 0)),
        ],
        out_specs=[
            pl.BlockSpec((_B, 1), lambda j: (0, 0)),
            pl.BlockSpec((_B, _C), lambda j: (0, 0)),
        ],
        out_shape=[
            jax.ShapeDtypeStruct((_B, 1), jnp.float32),
            jax.ShapeDtypeStruct((_B, _C), jnp.int32),
        ],
        scratch_shapes=[pltpu.VMEM((_B, 1), jnp.float32)],
        compiler_params=pltpu.CompilerParams(
            dimension_semantics=("arbitrary",)),
    )(slog, m_col, g1.reshape(_B, _C * 128), idx1)

    g2 = _sc_gather(slog_lin.reshape(_B * _NG16, 16), idx2.reshape(_B * _C),
                    _B * _C, 16, _C, 32, tc_tiling=False)

    out, tok = pl.pallas_call(
        _k5_body,
        grid=(_NCH,),
        in_specs=[
            pl.BlockSpec((_B, _CH), lambda j: (0, j)),
            pl.BlockSpec((_B, _CH), lambda j: (0, j)),
            pl.BlockSpec((_B, 1), lambda j: (0, 0)),
            pl.BlockSpec((_B, 1), lambda j: (0, 0)),
            pl.BlockSpec((_B, _C * 16), lambda j: (0, 0)),
            pl.BlockSpec((_B, _C), lambda j: (0, 0)),
            pl.BlockSpec((_B, 1), lambda j: (0, 0)),
            pl.BlockSpec((_B, 1), lambda j: (0, 0)),
        ],
        out_specs=[
            pl.BlockSpec((_B, _CH), lambda j: (0, j)),
            pl.BlockSpec((_B, 1), lambda j: (0, 0)),
        ],
        out_shape=[
            jax.ShapeDtypeStruct((_B, _VOCAB), jnp.float32),
            jax.ShapeDtypeStruct((_B, 1), jnp.int32),
        ],
        scratch_shapes=[pltpu.VMEM((_B, 1), jnp.float32),
                        pltpu.VMEM((_B, 1), jnp.int32),
                        pltpu.VMEM((_B, 1), jnp.float32),
                        pltpu.VMEM((_B, 1), jnp.int32),
                        pltpu.VMEM((_B, 1), jnp.float32)],
        compiler_params=pltpu.CompilerParams(
            dimension_semantics=("arbitrary",)),
    )(slog, u, m_col, s_col, g2.reshape(_B, _C * 16), idx2, tk_col, tp_col)

    return tok.reshape(_B), out


# speedup vs baseline: 1.0190x; 1.0190x over previous
import functools

import numpy as np
import jax
import jax.numpy as jnp
from jax import lax
from jax.experimental import pallas as pl
from jax.experimental.pallas import tpu as pltpu
from jax.experimental.pallas import tpu_sc as plsc

_VOCAB = 100000
_VP = 100352
_B = 64
_D = 1024
_CH = 3584
_NCH = 28
_NC128 = 784
_NG16 = 6272
_C = 112
_EPS = 1e-5
_NEG = float("-inf")


def _threefry_uniform(n):
    rot = [[13, 15, 26, 6], [17, 29, 16, 24]]
    k1, k2 = np.uint32(0), np.uint32(1)
    ks = [k1, k2, np.uint32(k1 ^ k2 ^ np.uint32(0x1BD11BDA))]

    def rotl(v, r):
        return ((v << np.uint32(r)) | (v >> np.uint32(32 - r))).astype(np.uint32)

    x0 = np.zeros(n, np.uint32)
    x1 = np.arange(n, dtype=np.uint32)
    x0 = (x0 + ks[0]).astype(np.uint32)
    x1 = (x1 + ks[1]).astype(np.uint32)
    for i in range(5):
        for r in rot[i % 2]:
            x0 = (x0 + x1).astype(np.uint32)
            x1 = rotl(x1, r)
            x1 = (x1 ^ x0).astype(np.uint32)
        x0 = (x0 + ks[(i + 1) % 3]).astype(np.uint32)
        x1 = (x1 + ks[(i + 2) % 3] + np.uint32(i + 1)).astype(np.uint32)
    bits = (x0 ^ x1).astype(np.uint32)
    fb = (bits >> np.uint32(9)) | np.uint32(0x3F800000)
    fl = fb.view(np.float32) - np.float32(1.0)
    tiny = np.float32(np.finfo(np.float32).tiny)
    return np.maximum(tiny, (fl + tiny).astype(np.float32))


_U_CONST = np.pad(
    _threefry_uniform(_B * _VOCAB).reshape(_B, _VOCAB),
    ((0, 0), (0, _VP - _VOCAB)), constant_values=0.5)


def _sc_gather(table, idx, n_idx, width, rows_per_batch, n_workers,
               tc_tiling=True):
    mesh = plsc.VectorSubcoreMesh(core_axis_name="c", subcore_axis_name="s")
    per_w = n_idx // n_workers
    n_batch = per_w // rows_per_batch

    @functools.partial(
        pl.kernel, mesh=mesh,
        out_type=jax.ShapeDtypeStruct((n_idx, width), jnp.float32),
        scratch_types=[
            pltpu.VMEM((rows_per_batch,), jnp.int32),
            pltpu.VMEM((rows_per_batch, width), jnp.float32),
            pltpu.SemaphoreType.DMA,
        ],
        compiler_params=pltpu.CompilerParams(use_tc_tiling_on_sc=tc_tiling),
    )
    def k(table_hbm, idx_hbm, out_hbm, idx_v, rows_v, sem):
        wid = lax.axis_index("s") * 2 + lax.axis_index("c")

        @pl.when(wid < n_workers)
        def _():
            base = wid * per_w
            for b in range(n_batch):
                off = base + b * rows_per_batch
                pltpu.sync_copy(idx_hbm.at[pl.ds(off, rows_per_batch)], idx_v)
                pltpu.async_copy(table_hbm.at[idx_v], rows_v, sem).wait()
                pltpu.sync_copy(rows_v, out_hbm.at[pl.ds(off, rows_per_batch)])

    return k(table, idx)


def _k1_body(h_ref, emb_ref, t_ref, slog_ref, idx1_ref, m_ref, macc):
    lg = lax.dot_general(h_ref[...], emb_ref[...], (((1,), (1,)), ((), ())),
                         preferred_element_type=jnp.float32)
    slog = lg / t_ref[...]
    j = pl.program_id(0)
    gi = j * _CH + lax.broadcasted_iota(jnp.int32, (_B, _CH), 1)
    slog = jnp.where(gi < _VOCAB, slog, _NEG)
    slog_ref[...] = slog

    @pl.when(j == 0)
    def _():
        macc[...] = jnp.full((_B, 896), _NEG, jnp.float32)

    cm = jnp.max(slog.reshape(_B, 28, 128), axis=2)
    placed = pltpu.roll(
        jnp.concatenate(
            [cm, jnp.full((_B, 896 - 28), _NEG, jnp.float32)], axis=1),
        j * 28, axis=1)
    macc[...] = jnp.maximum(macc[...], placed)

    @pl.when(j == _NCH - 1)
    def _():
        vals = macc[...]
        m_ref[...] = jnp.max(vals, axis=1, keepdims=True)
        iota = lax.broadcasted_iota(jnp.int32, (_B, 896), 1)
        ci = lax.broadcasted_iota(jnp.int32, (_B, _C), 1)
        rbase = lax.broadcasted_iota(jnp.int32, (_B, 1), 0) * _NC128

        def body(i, carry):
            vals, idx1 = carry
            m = jnp.max(vals, axis=1, keepdims=True)
            idx = jnp.min(jnp.where(vals >= m, iota, jnp.int32(1 << 30)),
                          axis=1, keepdims=True)
            idx1 = jnp.where(ci == i, rbase + idx, idx1)
            vals = jnp.where(iota == idx, _NEG, vals)
            return vals, idx1

        _, idx1 = lax.fori_loop(0, _C, body,
                                (vals, jnp.zeros((_B, _C), jnp.int32)))
        idx1_ref[...] = idx1


def _k3_body(g1_ref, cid_ref, idx2_ref):
    x = g1_ref[...]
    m16 = jnp.max(x.reshape(_B, _C * 8, 16), axis=-1)
    cid = cid_ref[...] % _NC128
    iota = lax.broadcasted_iota(jnp.int32, (_B, _C * 8), 1)
    ci = lax.broadcasted_iota(jnp.int32, (_B, _C), 1)
    lane = lax.broadcasted_iota(jnp.int32, (_B, _C), 1)
    rbase = lax.broadcasted_iota(jnp.int32, (_B, 1), 0) * _NG16

    def body(i, carry):
        vals, idx2 = carry
        m = jnp.max(vals, axis=1, keepdims=True)
        g = jnp.min(jnp.where(vals >= m, iota, jnp.int32(1 << 30)),
                    axis=1, keepdims=True)
        c = g >> 3
        sub = g & 7
        cidg = jnp.max(jnp.where(lane == c, cid, jnp.int32(-1)),
                       axis=1, keepdims=True)
        idx2 = jnp.where(ci == i, rbase + cidg * 8 + sub, idx2)
        vals = jnp.where(iota == g, _NEG, vals)
        return vals, idx2

    _, idx2 = lax.fori_loop(0, _C, body, (m16, jnp.zeros((_B, _C), jnp.int32)))
    idx2_ref[...] = idx2


def _ks_body(slog_ref, m_ref, s_ref, acc):
    j = pl.program_id(0)
    e = jnp.exp(slog_ref[...] - m_ref[...])
    ps = jnp.sum(e, axis=1, keepdims=True)
    prev = jnp.where(j == 0, 0.0, acc[...])
    acc[...] = prev + ps
    s_ref[...] = acc[...]


def _k4_body(g2_ref, idx2_ref, m_ref, s_ref, tk_ref, tp_ref,
             lcut_ref, icut_ref, tot_ref):
    vals0 = g2_ref[...]
    base16 = (idx2_ref[...] % _NG16) * 16
    vidx = (jnp.broadcast_to(base16[:, :, None], (_B, _C, 16)).reshape(_B, _C * 16)
            + (lax.broadcasted_iota(jnp.int32, (_B, _C * 16), 1) & 15))
    ci = lax.broadcasted_iota(jnp.int32, (_B, _C), 1)

    def body(i, carry):
        vals, cv, cvi = carry
        m = jnp.max(vals, axis=1, keepdims=True)
        vi = jnp.min(jnp.where(vals >= m, vidx, jnp.int32(1 << 30)),
                     axis=1, keepdims=True)
        cv = jnp.where(ci == i, m, cv)
        cvi = jnp.where(ci == i, vi, cvi)
        vals = jnp.where((vals >= m) & (vidx == vi), _NEG, vals)
        return vals, cv, cvi

    _, cv, cvi = lax.fori_loop(
        0, _C, body,
        (vals0, jnp.full((_B, _C), _NEG, jnp.float32),
         jnp.zeros((_B, _C), jnp.int32)))

    p = jnp.exp(cv - m_ref[...]) / s_ref[...]
    tkm1 = tk_ref[...] - 1
    thresh = jnp.max(jnp.where(ci == tkm1, p, _NEG), axis=1, keepdims=True)
    sel = p >= thresh
    sp = jnp.where(sel, p, 0.0)
    cum = sp
    for d in (1, 2, 4, 8, 16, 32, 64):
        cum = cum + jnp.concatenate(
            [jnp.zeros((_B, d), jnp.float32), cum[:, :_C - d]], axis=1)
    shifted_cum = jnp.concatenate(
        [jnp.zeros((_B, 1), jnp.float32), cum[:, :_C - 1]], axis=1)
    kept = sel & jnp.logical_not(shifted_cum > tp_ref[...])
    tot_ref[...] = jnp.sum(jnp.where(kept, p, 0.0), axis=1, keepdims=True)
    lastpos = jnp.max(jnp.where(kept, ci, -1), axis=1, keepdims=True)
    lcut_ref[...] = jnp.max(jnp.where(ci == lastpos, cv, _NEG),
                            axis=1, keepdims=True)
    icut_ref[...] = jnp.max(jnp.where(ci == lastpos, cvi, -1),
                            axis=1, keepdims=True)


def _k5_body(slog_ref, u_ref, m_ref, s_ref, lcut_ref, icut_ref, tot_ref,
             out_ref, tok_ref, bestv, besti):
    j = pl.program_id(0)
    x = slog_ref[...]
    gi = j * _CH + lax.broadcasted_iota(jnp.int32, (_B, _CH), 1)
    valid = gi < _VOCAB
    p = jnp.exp(x - m_ref[...]) / s_ref[...]
    keep = (x > lcut_ref[...]) | ((x == lcut_ref[...]) & (gi <= icut_ref[...]))
    outv = jnp.where(keep & valid, p / tot_ref[...], 0.0)
    out_ref[...] = outv

    g = -jnp.log(-jnp.log(u_ref[...]))
    val = jnp.where(valid, jnp.log(outv + 1e-20) + g, _NEG)
    mv = jnp.max(val, axis=1, keepdims=True)
    mi = jnp.min(jnp.where(val >= mv, gi, jnp.int32(1 << 30)),
                 axis=1, keepdims=True)
    pv = jnp.where(j == 0, _NEG, bestv[...])
    pi = jnp.where(j == 0, 0, besti[...])
    upd = mv > pv
    bestv[...] = jnp.where(upd, mv, pv)
    besti[...] = jnp.where(upd, mi, pi)
    tok_ref[...] = besti[...]


def kernel(hidden_states, embedding, last_token_indices, temperatures,
           top_ps, top_ks):
    lti = last_token_indices.astype(jnp.int32)
    temps_col = jnp.where(temperatures < _EPS, 1.0, temperatures)[:, None]
    tk_col = jnp.clip(top_ks, 1, _VOCAB).astype(jnp.int32)[:, None]
    tp_col = top_ps[:, None]
    u = jnp.asarray(_U_CONST)

    h = _sc_gather(hidden_states, lti, _B, _D, 8, 8)

    slog, idx1, m_col = pl.pallas_call(
        _k1_body,
        grid=(_NCH,),
        in_specs=[
            pl.BlockSpec((_B, _D), lambda j: (0, 0)),
            pl.BlockSpec((_CH, _D), lambda j: (j, 0)),
            pl.BlockSpec((_B, 1), lambda j: (0, 0)),
        ],
        out_specs=[
            pl.BlockSpec((_B, _CH), lambda j: (0, j)),
            pl.BlockSpec((_B, _C), lambda j: (0, 0)),
            pl.BlockSpec((_B, 1), lambda j: (0, 0)),
        ],
        out_shape=[
            jax.ShapeDtypeStruct((_B, _VP), jnp.float32),
            jax.ShapeDtypeStruct((_B, _C), jnp.int32),
            jax.ShapeDtypeStruct((_B, 1), jnp.float32),
        ],
        scratch_shapes=[pltpu.VMEM((_B, 896), jnp.float32)],
        compiler_params=pltpu.CompilerParams(
            dimension_semantics=("arbitrary",)),
    )(h, embedding, temps_col)

    s_col = pl.pallas_call(
        _ks_body,
        grid=(_NCH,),
        in_specs=[
            pl.BlockSpec((_B, _CH), lambda j: (0, j)),
            pl.BlockSpec((_B, 1), lambda j: (0, 0)),
        ],
        out_specs=pl.BlockSpec((_B, 1), lambda j: (0, 0)),
        out_shape=jax.ShapeDtypeStruct((_B, 1), jnp.float32),
        scratch_shapes=[pltpu.VMEM((_B, 1), jnp.float32)],
        compiler_params=pltpu.CompilerParams(
            dimension_semantics=("arbitrary",)),
    )(slog, m_col)

    slog_lin = slog.reshape(_B * _NC128, 128)
    g1 = _sc_gather(slog_lin, idx1.reshape(_B * _C),
                    _B * _C, 128, _C, 32)

    idx2 = pl.pallas_call(
        _k3_body,
        out_shape=jax.ShapeDtypeStruct((_B, _C), jnp.int32),
    )(g1.reshape(_B, _C * 128), idx1)

    g2 = _sc_gather(slog_lin.reshape(_B * _NG16, 16), idx2.reshape(_B * _C),
                    _B * _C, 16, _C, 32, tc_tiling=False)

    lcut, icut, tot = pl.pallas_call(
        _k4_body,
        out_shape=[
            jax.ShapeDtypeStruct((_B, 1), jnp.float32),
            jax.ShapeDtypeStruct((_B, 1), jnp.int32),
            jax.ShapeDtypeStruct((_B, 1), jnp.float32),
        ],
    )(g2.reshape(_B, _C * 16), idx2, m_col, s_col, tk_col, tp_col)

    out, tok = pl.pallas_call(
        _k5_body,
        grid=(_NCH,),
        in_specs=[
            pl.BlockSpec((_B, _CH), lambda j: (0, j)),
            pl.BlockSpec((_B, _CH), lambda j: (0, j)),
            pl.BlockSpec((_B, 1), lambda j: (0, 0)),
            pl.BlockSpec((_B, 1), lambda j: (0, 0)),
            pl.BlockSpec((_B, 1), lambda j: (0, 0)),
            pl.BlockSpec((_B, 1), lambda j: (0, 0)),
            pl.BlockSpec((_B, 1), lambda j: (0, 0)),
        ],
        out_specs=[
            pl.BlockSpec((_B, _CH), lambda j: (0, j)),
            pl.BlockSpec((_B, 1), lambda j: (0, 0)),
        ],
        out_shape=[
            jax.ShapeDtypeStruct((_B, _VOCAB), jnp.float32),
            jax.ShapeDtypeStruct((_B, 1), jnp.int32),
        ],
        scratch_shapes=[pltpu.VMEM((_B, 1), jnp.float32),
                        pltpu.VMEM((_B, 1), jnp.int32)],
        compiler_params=pltpu.CompilerParams(
            dimension_semantics=("arbitrary",)),
    )(slog, u, m_col, s_col, lcut, icut, tot)

    return tok.reshape(_B), out
